# baseline (speedup 1.0000x reference)
import functools

import jax
import jax.numpy as jnp
from jax import lax
from jax.experimental import pallas as pl
from jax.experimental.pallas import tpu as pltpu
from jax.experimental.pallas import tpu_sc as plsc

KNN_K = 16
LANES = 16


def _build_sc_kernel(B, P, N, D):
    NC, NS = 2, 16
    NW = NC * NS
    QW = (B * P) // NW
    WPB = NW // B
    NB = N // LANES
    OW = 2 * D * KNN_K
    mesh = plsc.VectorSubcoreMesh(
        core_axis_name="c", subcore_axis_name="s",
        num_cores=NC, num_subcores=NS)

    @functools.partial(
        pl.kernel,
        out_type=jax.ShapeDtypeStruct((B, P * OW), jnp.float32),
        mesh=mesh,
        compiler_params=pltpu.CompilerParams(
            needs_layout_passes=False, use_tc_tiling_on_sc=False),
        scratch_types=[
            pltpu.VMEM((3 * N,), jnp.float32),
            pltpu.VMEM((QW * 3 * LANES,), jnp.float32),
            pltpu.VMEM((QW * D,), jnp.float32),
            pltpu.VMEM((N,), jnp.float32),
            pltpu.VMEM((LANES,), jnp.float32),
            pltpu.VMEM((N + LANES,), jnp.float32),
            pltpu.VMEM((N + LANES,), jnp.int32),
            pltpu.VMEM((2, KNN_K, D), jnp.float32),
            pltpu.VMEM((2 * OW,), jnp.float32),
            pltpu.SemaphoreType.DMA,
            pltpu.SemaphoreType.DMA,
        ],
    )
    def knn_combine(reft_hbm, qb_hbm, prev_hbm, cur_hbm, out_hbm,
                    refv, qv, curv, dists, rotv, bufd, bufi, rowsv,
                    outv, sem_g, sem_o):
        wid = lax.axis_index("s") * NC + lax.axis_index("c")
        b = wid // WPB
        q0 = (wid % WPB) * QW

        pltpu.sync_copy(reft_hbm.at[b], refv)
        pltpu.sync_copy(qb_hbm.at[b, pl.ds(q0 * 3 * LANES, QW * 3 * LANES)],
                        qv)
        pltpu.sync_copy(cur_hbm.at[b, pl.ds(q0 * D, QW * D)], curv)

        iota = lax.iota(jnp.int32, LANES)
        infv = jnp.full((LANES,), jnp.inf, jnp.float32)
        zerov = jnp.zeros((LANES,), jnp.int32)

        def compute_topk(q):
            qbase = q * 3 * LANES
            qx = qv[pl.ds(qbase, LANES)]
            qy = qv[pl.ds(qbase + LANES, LANES)]
            qz = qv[pl.ds(qbase + 2 * LANES, LANES)]

            def pass_a(g, macc):
                for u in range(8):
                    base = (g * 8 + u) * LANES
                    rx = refv[pl.ds(base, LANES)]
                    ry = refv[pl.ds(N + base, LANES)]
                    rz = refv[pl.ds(2 * N + base, LANES)]
                    dx = rx - qx
                    dy = ry - qy
                    dz = rz - qz
                    d = dx * dx + dy * dy + dz * dz
                    dists[pl.ds(base, LANES)] = d
                    macc = jnp.minimum(macc, d)
                return macc

            macc = lax.fori_loop(0, NB // 8, pass_a, infv)
            tvec = macc
            for s in (8, 4, 2, 1):
                rotv[...] = tvec
                tvec = jnp.maximum(
                    tvec, plsc.load_gather(rotv, [(iota + s) & (LANES - 1)]))

            def pass_b(g, off):
                for u in range(8):
                    base = (g * 8 + u) * LANES
                    d = dists[pl.ds(base, LANES)]
                    m = d <= tvec
                    pos = off + plsc.cumsum(m.astype(jnp.int32)) - 1
                    plsc.store_scatter(bufd, [pos], d, mask=m)
                    plsc.store_scatter(bufi, [pos], base + iota, mask=m)
                    off = off + plsc.all_reduce_population_count(m)
                return off

            offv = lax.fori_loop(0, NB // 8, pass_b, zerov)
            num_cand = jnp.sum(jnp.where(iota == 0, offv, 0))
            padpos = offv + iota
            plsc.store_scatter(bufd, [padpos], infv)
            plsc.store_scatter(bufi, [padpos], zerov)

            topd, topi = plsc.sort_key_val(bufd[pl.ds(0, LANES)],
                                           bufi[pl.ds(0, LANES)])

            def merge(j, ti):
                t, i = ti
                dj, ij = plsc.sort_key_val(bufd[pl.ds(j * LANES, LANES)],
                                           bufi[pl.ds(j * LANES, LANES)])
                dr = lax.rev(dj, (0,))
                ir = lax.rev(ij, (0,))
                keep = t <= dr
                st, si = plsc.sort_key_val(jnp.where(keep, t, dr),
                                           jnp.where(keep, i, ir))
                return (st, si)

            nblk = (num_cand + (LANES - 1)) // LANES
            _, topi = lax.fori_loop(1, nblk, merge, (topd, topi))
            return topi

        def assemble(qq):
            s1 = qq & 1
            sv = jnp.full((LANES,), s1, jnp.int32)
            obase = s1 * OW
            for c in range(D // LANES):
                cu = curv[pl.ds(qq * D + c * LANES, LANES)]
                col = c * LANES + iota
                for k in range(KNN_K):
                    row = plsc.load_gather(
                        rowsv, [sv, jnp.full((LANES,), k, jnp.int32), col])
                    outv[pl.ds(obase + k * 2 * D + c * LANES, LANES)] = (
                        row - cu)
                    outv[pl.ds(obase + k * 2 * D + D + c * LANES, LANES)] = cu
            pltpu.async_copy(outv.at[pl.ds(obase, OW)],
                             out_hbm.at[b, pl.ds((q0 + qq) * OW, OW)], sem_o)

        def pipe(q, carry):
            @pl.when(q < QW)
            def _():
                topi = compute_topk(q)
                pltpu.async_copy(prev_hbm.at[topi], rowsv.at[q & 1], sem_g)

            @pl.when(q > 0)
            def _():
                qq = q - 1
                pltpu.make_async_copy(prev_hbm.at[pl.ds(0, KNN_K)],
                                      rowsv.at[qq & 1], sem_g).wait()

                @pl.when(q >= 3)
                def _():
                    pltpu.make_async_copy(
                        outv.at[pl.ds(0, OW)],
                        out_hbm.at[b, pl.ds(0, OW)], sem_o).wait()

                assemble(qq)
            return carry

        lax.fori_loop(0, QW + 1, pipe, 0)
        pltpu.make_async_copy(outv.at[pl.ds(0, OW)],
                              out_hbm.at[b, pl.ds(0, OW)], sem_o).wait()
        pltpu.make_async_copy(outv.at[pl.ds(0, OW)],
                              out_hbm.at[b, pl.ds(0, OW)], sem_o).wait()

    return knn_combine


def kernel(points_ref, points_query, previous_features, current_features):
    B, N, _ = points_ref.shape
    P = points_query.shape[1]
    D = previous_features.shape[-1]
    reft = jnp.transpose(points_ref, (0, 2, 1)).reshape(B, 3 * N)
    qb = jnp.broadcast_to(points_query[:, :, :, None],
                          (B, P, 3, LANES)).reshape(B, P * 3 * LANES)
    prev0 = previous_features[0]
    cur = current_features.reshape(B, P * D)
    fn = _build_sc_kernel(B, P, N, D)
    out = fn(reft, qb, prev0, cur)
    return out.reshape(B, P * KNN_K, 2 * D)


# speedup vs baseline: 2.8396x; 2.8396x over previous
import functools

import jax
import jax.numpy as jnp
from jax import lax
from jax.experimental import pallas as pl
from jax.experimental.pallas import tpu as pltpu
from jax.experimental.pallas import tpu_sc as plsc

KNN_K = 16
LANES = 16


def _build_sc_kernel(B, P, N, D):
    NC, NS = 2, 16
    NW = NC * NS
    QW = (B * P) // NW
    WPB = NW // B
    NB = N // LANES
    OW = 2 * D * KNN_K
    mesh = plsc.VectorSubcoreMesh(
        core_axis_name="c", subcore_axis_name="s",
        num_cores=NC, num_subcores=NS)

    @functools.partial(
        pl.kernel,
        out_type=jax.ShapeDtypeStruct((B, P * OW), jnp.float32),
        mesh=mesh,
        compiler_params=pltpu.CompilerParams(
            needs_layout_passes=False, use_tc_tiling_on_sc=False),
        scratch_types=[
            pltpu.VMEM((3 * N,), jnp.float32),
            pltpu.VMEM((QW * 3 * LANES,), jnp.float32),
            pltpu.VMEM((QW * D,), jnp.float32),
            pltpu.VMEM((N,), jnp.float32),
            pltpu.VMEM((LANES,), jnp.float32),
            pltpu.VMEM((N + LANES,), jnp.float32),
            pltpu.VMEM((N + LANES,), jnp.int32),
            pltpu.VMEM((2, KNN_K, D), jnp.float32),
            pltpu.VMEM((2 * OW,), jnp.float32),
            pltpu.SemaphoreType.DMA,
            pltpu.SemaphoreType.DMA,
        ],
    )
    def knn_combine(reft_hbm, qb_hbm, prev_hbm, cur_hbm, out_hbm,
                    refv, qv, curv, dists, rotv, bufd, bufi, rowsv,
                    outv, sem_g, sem_o):
        wid = lax.axis_index("s") * NC + lax.axis_index("c")
        b = wid // WPB
        q0 = (wid % WPB) * QW

        pltpu.sync_copy(reft_hbm.at[b], refv)
        pltpu.sync_copy(qb_hbm.at[b, pl.ds(q0 * 3 * LANES, QW * 3 * LANES)],
                        qv)
        pltpu.sync_copy(cur_hbm.at[b, pl.ds(q0 * D, QW * D)], curv)

        iota = lax.iota(jnp.int32, LANES)
        infv = jnp.full((LANES,), jnp.inf, jnp.float32)
        zerov = jnp.zeros((LANES,), jnp.int32)

        def compute_topk(q):
            qbase = q * 3 * LANES
            qx = qv[pl.ds(qbase, LANES)]
            qy = qv[pl.ds(qbase + LANES, LANES)]
            qz = qv[pl.ds(qbase + 2 * LANES, LANES)]

            def pass_a(blk, macc):
                base = blk * LANES
                rx = refv[pl.ds(base, LANES)]
                ry = refv[pl.ds(N + base, LANES)]
                rz = refv[pl.ds(2 * N + base, LANES)]
                dx = rx - qx
                dy = ry - qy
                dz = rz - qz
                d = dx * dx + dy * dy + dz * dz
                dists[pl.ds(base, LANES)] = d
                return jnp.minimum(macc, d)

            macc = plsc.parallel_loop(0, NB, unroll=8, carry=infv)(pass_a)
            tvec = macc
            for s in (8, 4, 2, 1):
                rotv[...] = tvec
                tvec = jnp.maximum(
                    tvec, plsc.load_gather(rotv, [(iota + s) & (LANES - 1)]))

            def pass_b(blk, off):
                base = blk * LANES
                d = dists[pl.ds(base, LANES)]
                m = d <= tvec
                pos = off + plsc.cumsum(m.astype(jnp.int32)) - 1
                plsc.store_scatter(bufd, [pos], d, mask=m)
                plsc.store_scatter(bufi, [pos], base + iota, mask=m)
                return off + plsc.all_reduce_population_count(m)

            offv = plsc.parallel_loop(0, NB, unroll=8, carry=zerov)(pass_b)
            num_cand = jnp.sum(jnp.where(iota == 0, offv, 0))
            padpos = offv + iota
            plsc.store_scatter(bufd, [padpos], infv)
            plsc.store_scatter(bufi, [padpos], zerov)

            topd, topi = plsc.sort_key_val(bufd[pl.ds(0, LANES)],
                                           bufi[pl.ds(0, LANES)])

            def merge(j, ti):
                t, i = ti
                dj, ij = plsc.sort_key_val(bufd[pl.ds(j * LANES, LANES)],
                                           bufi[pl.ds(j * LANES, LANES)])
                dr = lax.rev(dj, (0,))
                ir = lax.rev(ij, (0,))
                keep = t <= dr
                st, si = plsc.sort_key_val(jnp.where(keep, t, dr),
                                           jnp.where(keep, i, ir))
                return (st, si)

            nblk = (num_cand + (LANES - 1)) // LANES
            _, topi = lax.fori_loop(1, nblk, merge, (topd, topi))
            return topi

        def assemble(qq):
            s1 = qq & 1
            sv = jnp.full((LANES,), s1, jnp.int32)
            obase = s1 * OW
            for c in range(D // LANES):
                cu = curv[pl.ds(qq * D + c * LANES, LANES)]
                col = c * LANES + iota
                for k in range(KNN_K):
                    row = plsc.load_gather(
                        rowsv, [sv, jnp.full((LANES,), k, jnp.int32), col])
                    outv[pl.ds(obase + k * 2 * D + c * LANES, LANES)] = (
                        row - cu)
                    outv[pl.ds(obase + k * 2 * D + D + c * LANES, LANES)] = cu
            pltpu.async_copy(outv.at[pl.ds(obase, OW)],
                             out_hbm.at[b, pl.ds((q0 + qq) * OW, OW)], sem_o)

        def pipe(q, carry):
            @pl.when(q < QW)
            def _():
                topi = compute_topk(q)
                pltpu.async_copy(prev_hbm.at[topi], rowsv.at[q & 1], sem_g)

            @pl.when(q > 0)
            def _():
                qq = q - 1
                pltpu.make_async_copy(prev_hbm.at[pl.ds(0, KNN_K)],
                                      rowsv.at[qq & 1], sem_g).wait()

                @pl.when(q >= 3)
                def _():
                    pltpu.make_async_copy(
                        outv.at[pl.ds(0, OW)],
                        out_hbm.at[b, pl.ds(0, OW)], sem_o).wait()

                assemble(qq)
            return carry

        lax.fori_loop(0, QW + 1, pipe, 0)
        pltpu.make_async_copy(outv.at[pl.ds(0, OW)],
                              out_hbm.at[b, pl.ds(0, OW)], sem_o).wait()
        pltpu.make_async_copy(outv.at[pl.ds(0, OW)],
                              out_hbm.at[b, pl.ds(0, OW)], sem_o).wait()

    return knn_combine


def kernel(points_ref, points_query, previous_features, current_features):
    B, N, _ = points_ref.shape
    P = points_query.shape[1]
    D = previous_features.shape[-1]
    reft = jnp.transpose(points_ref, (0, 2, 1)).reshape(B, 3 * N)
    qb = jnp.broadcast_to(points_query[:, :, :, None],
                          (B, P, 3, LANES)).reshape(B, P * 3 * LANES)
    prev0 = previous_features[0]
    cur = current_features.reshape(B, P * D)
    fn = _build_sc_kernel(B, P, N, D)
    out = fn(reft, qb, prev0, cur)
    return out.reshape(B, P * KNN_K, 2 * D)


# speedup vs baseline: 2.8768x; 1.0131x over previous
import functools

import jax
import jax.numpy as jnp
from jax import lax
from jax.experimental import pallas as pl
from jax.experimental.pallas import tpu as pltpu
from jax.experimental.pallas import tpu_sc as plsc

KNN_K = 16
LANES = 16


def _build_sc_kernel(B, P, N, D):
    NC, NS = 2, 16
    NW = NC * NS
    QW = (B * P) // NW
    WPB = NW // B
    NB = N // LANES
    NP = QW // 2
    OW = 2 * D * KNN_K
    mesh = plsc.VectorSubcoreMesh(
        core_axis_name="c", subcore_axis_name="s",
        num_cores=NC, num_subcores=NS)

    @functools.partial(
        pl.kernel,
        out_type=jax.ShapeDtypeStruct((B, P * OW), jnp.float32),
        mesh=mesh,
        compiler_params=pltpu.CompilerParams(
            needs_layout_passes=False, use_tc_tiling_on_sc=False),
        scratch_types=[
            pltpu.VMEM((3 * N,), jnp.float32),
            pltpu.VMEM((QW * 3 * LANES,), jnp.float32),
            pltpu.VMEM((QW * D,), jnp.float32),
            pltpu.VMEM((2 * N,), jnp.float32),
            pltpu.VMEM((LANES,), jnp.float32),
            pltpu.VMEM((N + LANES,), jnp.float32),
            pltpu.VMEM((N + LANES,), jnp.int32),
            pltpu.VMEM((4, KNN_K, D), jnp.float32),
            pltpu.VMEM((4 * OW,), jnp.float32),
            pltpu.SemaphoreType.DMA,
            pltpu.SemaphoreType.DMA,
        ],
    )
    def knn_combine(reft_hbm, qb_hbm, prev_hbm, cur_hbm, out_hbm,
                    refv, qv, curv, dists, rotv, bufd, bufi, rowsv,
                    outv, sem_g, sem_o):
        wid = lax.axis_index("s") * NC + lax.axis_index("c")
        b = wid // WPB
        q0 = (wid % WPB) * QW

        pltpu.sync_copy(reft_hbm.at[b], refv)
        pltpu.sync_copy(qb_hbm.at[b, pl.ds(q0 * 3 * LANES, QW * 3 * LANES)],
                        qv)
        pltpu.sync_copy(cur_hbm.at[b, pl.ds(q0 * D, QW * D)], curv)

        iota = lax.iota(jnp.int32, LANES)
        infv = jnp.full((LANES,), jnp.inf, jnp.float32)
        zerov = jnp.zeros((LANES,), jnp.int32)

        def lane_max_splat(vec):
            for s in (8, 4, 2, 1):
                rotv[...] = vec
                vec = jnp.maximum(
                    vec, plsc.load_gather(rotv, [(iota + s) & (LANES - 1)]))
            return vec

        def select_topk(doff, tvec):
            def pass_b(blk, off):
                base = blk * LANES
                d = dists[pl.ds(doff + base, LANES)]
                m = d <= tvec
                pos = off + plsc.cumsum(m.astype(jnp.int32)) - 1
                plsc.store_scatter(bufd, [pos], d, mask=m)
                plsc.store_scatter(bufi, [pos], base + iota, mask=m)
                return off + plsc.all_reduce_population_count(m)

            offv = plsc.parallel_loop(0, NB, unroll=8, carry=zerov)(pass_b)
            num_cand = jnp.sum(jnp.where(iota == 0, offv, 0))
            padpos = offv + iota
            plsc.store_scatter(bufd, [padpos], infv)
            plsc.store_scatter(bufi, [padpos], zerov)

            topd, topi = plsc.sort_key_val(bufd[pl.ds(0, LANES)],
                                           bufi[pl.ds(0, LANES)])

            def merge(j, ti):
                t, i = ti
                dj, ij = plsc.sort_key_val(bufd[pl.ds(j * LANES, LANES)],
                                           bufi[pl.ds(j * LANES, LANES)])
                dr = lax.rev(dj, (0,))
                ir = lax.rev(ij, (0,))
                keep = t <= dr
                st, si = plsc.sort_key_val(jnp.where(keep, t, dr),
                                           jnp.where(keep, i, ir))
                return (st, si)

            nblk = (num_cand + (LANES - 1)) // LANES
            _, topi = lax.fori_loop(1, nblk, merge, (topd, topi))
            return topi

        def compute_pair(p):
            qa = 2 * p
            abase = qa * 3 * LANES
            qxa = qv[pl.ds(abase, LANES)]
            qya = qv[pl.ds(abase + LANES, LANES)]
            qza = qv[pl.ds(abase + 2 * LANES, LANES)]
            qxb = qv[pl.ds(abase + 3 * LANES, LANES)]
            qyb = qv[pl.ds(abase + 4 * LANES, LANES)]
            qzb = qv[pl.ds(abase + 5 * LANES, LANES)]

            def pass_a(blk, carry):
                macc_a, macc_b = carry
                base = blk * LANES
                rx = refv[pl.ds(base, LANES)]
                ry = refv[pl.ds(N + base, LANES)]
                rz = refv[pl.ds(2 * N + base, LANES)]
                dxa = rx - qxa
                dya = ry - qya
                dza = rz - qza
                da = dxa * dxa + dya * dya + dza * dza
                dists[pl.ds(base, LANES)] = da
                dxb = rx - qxb
                dyb = ry - qyb
                dzb = rz - qzb
                db = dxb * dxb + dyb * dyb + dzb * dzb
                dists[pl.ds(N + base, LANES)] = db
                return (jnp.minimum(macc_a, da), jnp.minimum(macc_b, db))

            macc_a, macc_b = plsc.parallel_loop(
                0, NB, unroll=8, carry=(infv, infv))(pass_a)
            ta = lane_max_splat(macc_a)
            tb = lane_max_splat(macc_b)

            topi_a = select_topk(0, ta)
            pltpu.async_copy(prev_hbm.at[topi_a], rowsv.at[qa & 3], sem_g)
            topi_b = select_topk(N, tb)
            pltpu.async_copy(prev_hbm.at[topi_b], rowsv.at[(qa + 1) & 3],
                             sem_g)

        def assemble(qq):
            s1 = qq & 3
            sv = jnp.full((LANES,), s1, jnp.int32)
            obase = s1 * OW
            for c in range(D // LANES):
                cu = curv[pl.ds(qq * D + c * LANES, LANES)]
                col = c * LANES + iota
                for k in range(KNN_K):
                    row = plsc.load_gather(
                        rowsv, [sv, jnp.full((LANES,), k, jnp.int32), col])
                    outv[pl.ds(obase + k * 2 * D + c * LANES, LANES)] = (
                        row - cu)
                    outv[pl.ds(obase + k * 2 * D + D + c * LANES, LANES)] = cu
            pltpu.async_copy(outv.at[pl.ds(obase, OW)],
                             out_hbm.at[b, pl.ds((q0 + qq) * OW, OW)], sem_o)

        def pipe(p, carry):
            @pl.when(p < NP)
            def _():
                compute_pair(p)

            @pl.when(p > 0)
            def _():
                qa = 2 * (p - 1)
                pltpu.make_async_copy(prev_hbm.at[pl.ds(0, KNN_K)],
                                      rowsv.at[qa & 3], sem_g).wait()
                pltpu.make_async_copy(prev_hbm.at[pl.ds(0, KNN_K)],
                                      rowsv.at[(qa + 1) & 3], sem_g).wait()

                @pl.when(p >= 3)
                def _():
                    for _ in range(2):
                        pltpu.make_async_copy(
                            outv.at[pl.ds(0, OW)],
                            out_hbm.at[b, pl.ds(0, OW)], sem_o).wait()

                assemble(qa)
                assemble(qa + 1)
            return carry

        lax.fori_loop(0, NP + 1, pipe, 0)
        for _ in range(4):
            pltpu.make_async_copy(outv.at[pl.ds(0, OW)],
                                  out_hbm.at[b, pl.ds(0, OW)], sem_o).wait()

    return knn_combine


def kernel(points_ref, points_query, previous_features, current_features):
    B, N, _ = points_ref.shape
    P = points_query.shape[1]
    D = previous_features.shape[-1]
    reft = jnp.transpose(points_ref, (0, 2, 1)).reshape(B, 3 * N)
    qb = jnp.broadcast_to(points_query[:, :, :, None],
                          (B, P, 3, LANES)).reshape(B, P * 3 * LANES)
    prev0 = previous_features[0]
    cur = current_features.reshape(B, P * D)
    fn = _build_sc_kernel(B, P, N, D)
    out = fn(reft, qb, prev0, cur)
    return out.reshape(B, P * KNN_K, 2 * D)


# speedup vs baseline: 2.8957x; 1.0066x over previous
import functools

import jax
import jax.numpy as jnp
from jax import lax
from jax.experimental import pallas as pl
from jax.experimental.pallas import tpu as pltpu
from jax.experimental.pallas import tpu_sc as plsc

KNN_K = 16
LANES = 16


def _build_sc_kernel(B, P, N, D):
    NC, NS = 2, 16
    NW = NC * NS
    QW = (B * P) // NW
    WPB = NW // B
    NB = N // LANES
    NP = QW // 2
    OW = 2 * D * KNN_K
    mesh = plsc.VectorSubcoreMesh(
        core_axis_name="c", subcore_axis_name="s",
        num_cores=NC, num_subcores=NS)

    @functools.partial(
        pl.kernel,
        out_type=jax.ShapeDtypeStruct((B, P * OW), jnp.float32),
        mesh=mesh,
        compiler_params=pltpu.CompilerParams(
            needs_layout_passes=False, use_tc_tiling_on_sc=False),
        scratch_types=[
            pltpu.VMEM((3 * N,), jnp.float32),
            pltpu.VMEM((QW * 3 * LANES,), jnp.float32),
            pltpu.VMEM((QW * D,), jnp.float32),
            pltpu.VMEM((2 * N,), jnp.float32),
            pltpu.VMEM((LANES,), jnp.float32),
            pltpu.VMEM((N + LANES,), jnp.float32),
            pltpu.VMEM((N + LANES,), jnp.int32),
            pltpu.VMEM((4 * KNN_K, D), jnp.float32),
            pltpu.VMEM((4 * OW,), jnp.float32),
            pltpu.SemaphoreType.DMA,
            pltpu.SemaphoreType.DMA,
        ],
    )
    def knn_combine(reft_hbm, qb_hbm, prev_hbm, cur_hbm, out_hbm,
                    refv, qv, curv, dists, rotv, bufd, bufi, rowsv,
                    outv, sem_g, sem_o):
        wid = lax.axis_index("s") * NC + lax.axis_index("c")
        b = wid // WPB
        q0 = (wid % WPB) * QW

        pltpu.sync_copy(reft_hbm.at[b], refv)
        pltpu.sync_copy(qb_hbm.at[b, pl.ds(q0 * 3 * LANES, QW * 3 * LANES)],
                        qv)
        pltpu.sync_copy(cur_hbm.at[b, pl.ds(q0 * D, QW * D)], curv)

        iota = lax.iota(jnp.int32, LANES)
        infv = jnp.full((LANES,), jnp.inf, jnp.float32)
        zerov = jnp.zeros((LANES,), jnp.int32)

        def lane_max_splat(vec):
            for s in (8, 4, 2, 1):
                rotv[...] = vec
                vec = jnp.maximum(
                    vec, plsc.load_gather(rotv, [(iota + s) & (LANES - 1)]))
            return vec

        def select_topk(doff, tvec):
            def pass_b(blk, off):
                base = blk * LANES
                d = dists[pl.ds(doff + base, LANES)]
                m = d <= tvec
                pos = off + plsc.cumsum(m.astype(jnp.int32)) - 1
                plsc.store_scatter(bufd, [pos], d, mask=m)
                plsc.store_scatter(bufi, [pos], base + iota, mask=m)
                return off + plsc.all_reduce_population_count(m)

            offv = plsc.parallel_loop(0, NB, unroll=8, carry=zerov)(pass_b)
            num_cand = jnp.sum(jnp.where(iota == 0, offv, 0))
            padpos = offv + iota
            plsc.store_scatter(bufd, [padpos], infv)
            plsc.store_scatter(bufi, [padpos], zerov)

            topd, topi = plsc.sort_key_val(bufd[pl.ds(0, LANES)],
                                           bufi[pl.ds(0, LANES)])

            def merge(j, ti):
                t, i = ti
                dj, ij = plsc.sort_key_val(bufd[pl.ds(j * LANES, LANES)],
                                           bufi[pl.ds(j * LANES, LANES)])
                dr = lax.rev(dj, (0,))
                ir = lax.rev(ij, (0,))
                keep = t <= dr
                st, si = plsc.sort_key_val(jnp.where(keep, t, dr),
                                           jnp.where(keep, i, ir))
                return (st, si)

            nblk = (num_cand + (LANES - 1)) // LANES
            _, topi = lax.fori_loop(1, nblk, merge, (topd, topi))
            return topi

        def compute_pair(p):
            qa = 2 * p
            abase = qa * 3 * LANES
            qxa = qv[pl.ds(abase, LANES)]
            qya = qv[pl.ds(abase + LANES, LANES)]
            qza = qv[pl.ds(abase + 2 * LANES, LANES)]
            qxb = qv[pl.ds(abase + 3 * LANES, LANES)]
            qyb = qv[pl.ds(abase + 4 * LANES, LANES)]
            qzb = qv[pl.ds(abase + 5 * LANES, LANES)]

            def pass_a(blk, carry):
                macc_a, macc_b = carry
                base = blk * LANES
                rx = refv[pl.ds(base, LANES)]
                ry = refv[pl.ds(N + base, LANES)]
                rz = refv[pl.ds(2 * N + base, LANES)]
                dxa = rx - qxa
                dya = ry - qya
                dza = rz - qza
                da = dxa * dxa + dya * dya + dza * dza
                dists[pl.ds(base, LANES)] = da
                dxb = rx - qxb
                dyb = ry - qyb
                dzb = rz - qzb
                db = dxb * dxb + dyb * dyb + dzb * dzb
                dists[pl.ds(N + base, LANES)] = db
                return (jnp.minimum(macc_a, da), jnp.minimum(macc_b, db))

            macc_a, macc_b = plsc.parallel_loop(
                0, NB, unroll=8, carry=(infv, infv))(pass_a)
            ta = lane_max_splat(macc_a)
            tb = lane_max_splat(macc_b)

            topi_a = select_topk(0, ta)
            pltpu.async_copy(
                prev_hbm.at[topi_a],
                rowsv.at[pl.ds((qa & 3) * KNN_K, KNN_K)], sem_g)
            topi_b = select_topk(N, tb)
            pltpu.async_copy(
                prev_hbm.at[topi_b],
                rowsv.at[pl.ds(((qa + 1) & 3) * KNN_K, KNN_K)], sem_g)

        def assemble(qq):
            s1 = qq & 3
            rbase = s1 * KNN_K
            obase = s1 * OW
            for c in range(D // LANES):
                cu = curv[pl.ds(qq * D + c * LANES, LANES)]
                col = c * LANES + iota
                for k in range(KNN_K):
                    row = plsc.load_gather(
                        rowsv, [jnp.full((LANES,), rbase + k, jnp.int32), col])
                    outv[pl.ds(obase + k * 2 * D + c * LANES, LANES)] = (
                        row - cu)
                    outv[pl.ds(obase + k * 2 * D + D + c * LANES, LANES)] = cu
            pltpu.async_copy(outv.at[pl.ds(obase, OW)],
                             out_hbm.at[b, pl.ds((q0 + qq) * OW, OW)], sem_o)

        def pipe(p, carry):
            @pl.when(p < NP)
            def _():
                compute_pair(p)

            @pl.when(p > 0)
            def _():
                qa = 2 * (p - 1)
                pltpu.make_async_copy(
                    prev_hbm.at[pl.ds(0, KNN_K)],
                    rowsv.at[pl.ds((qa & 3) * KNN_K, KNN_K)],
                    sem_g).wait()
                pltpu.make_async_copy(
                    prev_hbm.at[pl.ds(0, KNN_K)],
                    rowsv.at[pl.ds(((qa + 1) & 3) * KNN_K, KNN_K)],
                    sem_g).wait()

                @pl.when(p >= 3)
                def _():
                    for _ in range(2):
                        pltpu.make_async_copy(
                            outv.at[pl.ds(0, OW)],
                            out_hbm.at[b, pl.ds(0, OW)], sem_o).wait()

                assemble(qa)
                assemble(qa + 1)
            return carry

        lax.fori_loop(0, NP + 1, pipe, 0)
        for _ in range(4):
            pltpu.make_async_copy(outv.at[pl.ds(0, OW)],
                                  out_hbm.at[b, pl.ds(0, OW)], sem_o).wait()

    return knn_combine


def kernel(points_ref, points_query, previous_features, current_features):
    B, N, _ = points_ref.shape
    P = points_query.shape[1]
    D = previous_features.shape[-1]
    reft = jnp.transpose(points_ref, (0, 2, 1)).reshape(B, 3 * N)
    qb = jnp.broadcast_to(points_query[:, :, :, None],
                          (B, P, 3, LANES)).reshape(B, P * 3 * LANES)
    prev0 = previous_features[0]
    cur = current_features.reshape(B, P * D)
    fn = _build_sc_kernel(B, P, N, D)
    out = fn(reft, qb, prev0, cur)
    return out.reshape(B, P * KNN_K, 2 * D)
